# baseline (speedup 1.0000x reference)
import functools

import jax
import jax.numpy as jnp
from jax import lax
from jax.experimental import pallas as pl
from jax.experimental.pallas import tpu as pltpu
from jax.experimental.pallas import tpu_sc as plsc

_NC = 2
_NS = 16
_NW = _NC * _NS
_CHUNK = 256


def _make_gather(n_total: int, n_per_w: int, dim: int):
    mesh = plsc.VectorSubcoreMesh(core_axis_name="c", subcore_axis_name="s")
    n_chunks = n_per_w // _CHUNK

    @functools.partial(
        pl.kernel,
        mesh=mesh,
        out_type=jax.ShapeDtypeStruct((n_total, dim), jnp.float32),
        scratch_types=[
            pltpu.VMEM((n_per_w,), jnp.int32),
            pltpu.VMEM((_CHUNK,), jnp.int32),
            pltpu.VMEM((_CHUNK, 4 * dim), jnp.float32),
            pltpu.VMEM((_CHUNK, dim), jnp.float32),
            pltpu.SemaphoreType.DMA,
        ],
    )
    def gather_kernel(idx_hbm, table_hbm, out_hbm, idx_v, idx4_v, buf_v,
                      outb_v, sem):
        wid = lax.axis_index("s") * _NC + lax.axis_index("c")
        base = wid * n_per_w
        pltpu.sync_copy(idx_hbm.at[pl.ds(base, n_per_w)], idx_v)

        def do_chunk(c, _):
            cb = c * _CHUNK

            def prep(i, _):
                v = idx_v[pl.ds(cb + i * 16, 16)]
                idx4_v[pl.ds(i * 16, 16)] = v >> 2
                return 0

            lax.fori_loop(0, _CHUNK // 16, prep, 0)
            pltpu.async_copy(table_hbm.at[idx4_v], buf_v, sem).wait()

            def compact(g, _):
                rb = g * 16
                off = (idx_v[pl.ds(cb + rb, 16)] & 3) * dim
                for j in range(16):
                    oj = off[j]
                    outb_v[rb + j, pl.ds(0, 16)] = buf_v[rb + j, pl.ds(oj, 16)]
                    outb_v[rb + j, pl.ds(16, 16)] = (
                        buf_v[rb + j, pl.ds(oj + 16, 16)])
                return 0

            lax.fori_loop(0, _CHUNK // 16, compact, 0)
            pltpu.sync_copy(outb_v, out_hbm.at[pl.ds(base + cb, _CHUNK)])
            return 0

        lax.fori_loop(0, n_chunks, do_chunk, 0)

    return gather_kernel


def kernel(inputs, embedding):
    batch, fields = inputs.shape
    vocab, dim = embedding.shape
    n_total = batch * fields
    assert n_total % (_NW * _CHUNK) == 0
    n_per_w = n_total // _NW
    flat_idx = inputs.reshape(n_total).astype(jnp.int32)
    table = embedding.reshape(vocab // 4, 4 * dim)
    out = _make_gather(n_total, n_per_w, dim)(flat_idx, table)
    return out.reshape(batch, fields, dim)


# speedup vs baseline: 1.0618x; 1.0618x over previous
import functools

import jax
import jax.numpy as jnp
from jax import lax
from jax.experimental import pallas as pl
from jax.experimental.pallas import tpu as pltpu
from jax.experimental.pallas import tpu_sc as plsc

_NC = 2
_NS = 16
_NW = _NC * _NS


def _make_gather(fields: int, batch: int, dim: int):
    mesh = plsc.VectorSubcoreMesh(core_axis_name="c", subcore_axis_name="s")
    nb = batch // _NW

    @functools.partial(
        pl.kernel,
        mesh=mesh,
        out_type=jax.ShapeDtypeStruct((fields, dim, batch), jnp.float32),
        compiler_params=pltpu.CompilerParams(use_tc_tiling_on_sc=False,
                                             needs_layout_passes=False),
        scratch_types=[
            pltpu.VMEM((fields, nb), jnp.int32),
            pltpu.VMEM((fields * nb, dim), jnp.float32),
            pltpu.VMEM((dim, nb), jnp.float32),
            pltpu.VMEM((dim, nb), jnp.float32),
            pltpu.SemaphoreType.DMA,
            pltpu.SemaphoreType.DMA,
            pltpu.SemaphoreType.DMA,
        ],
    )
    def gather_kernel(idx_hbm, table_hbm, out_hbm, idx_v, rows_v,
                      slab0, slab1, gsem, wsem0, wsem1):
        wid = lax.axis_index("s") * _NC + lax.axis_index("c")
        b0 = wid * nb
        pltpu.sync_copy(idx_hbm.at[:, pl.ds(b0, nb)], idx_v)
        for f in range(fields):
            pltpu.async_copy(table_hbm.at[idx_v.at[f]],
                             rows_v.at[pl.ds(f * nb, nb)], gsem)
        pltpu.make_async_copy(table_hbm.at[pl.ds(0, fields * nb)], rows_v,
                              gsem).wait()
        slabs = (slab0, slab1)
        wsems = (wsem0, wsem1)
        lanes = lax.iota(jnp.int32, 16)

        def transpose_field(f, slab):
            def g_body(g, _):
                row16 = f * nb + g * 16 + lanes
                for c in range(dim):
                    col16 = jnp.full((16,), c, dtype=jnp.int32)
                    vals = plsc.load_gather(rows_v, [row16, col16])
                    slab[c, pl.ds(g * 16, 16)] = vals
                return 0

            lax.fori_loop(0, nb // 16, g_body, 0)

        def writeback_field(f, slab, wsem):
            def c_body(c, _):
                pltpu.async_copy(slab.at[c], out_hbm.at[f, c, pl.ds(b0, nb)],
                                 wsem)
                return 0

            lax.fori_loop(0, dim, c_body, 0)

        def drain_wb(slab, wsem):
            pltpu.make_async_copy(out_hbm.at[0, :, pl.ds(0, nb)], slab,
                                  wsem).wait()

        for f in range(fields):
            p = f % 2
            if f >= 2:
                drain_wb(slabs[p], wsems[p])
            transpose_field(f, slabs[p])
            writeback_field(f, slabs[p], wsems[p])
        drain_wb(slabs[0], wsems[0])
        drain_wb(slabs[1], wsems[1])

    return gather_kernel


def kernel(inputs, embedding):
    batch, fields = inputs.shape
    vocab, dim = embedding.shape
    assert batch % _NW == 0
    idx_t = inputs.T.astype(jnp.int32)
    out_t = _make_gather(fields, batch, dim)(idx_t, embedding)
    return out_t.transpose(2, 0, 1)


# speedup vs baseline: 1.0696x; 1.0074x over previous
import functools

import jax
import jax.numpy as jnp
from jax import lax
from jax.experimental import pallas as pl
from jax.experimental.pallas import tpu as pltpu
from jax.experimental.pallas import tpu_sc as plsc

_NC = 2
_NS = 16
_NW = _NC * _NS


def _make_gather(fields: int, batch: int, dim: int):
    mesh = plsc.VectorSubcoreMesh(core_axis_name="c", subcore_axis_name="s")
    nb = batch // _NW

    @functools.partial(
        pl.kernel,
        mesh=mesh,
        out_type=jax.ShapeDtypeStruct((fields, dim, batch), jnp.float32),
        compiler_params=pltpu.CompilerParams(use_tc_tiling_on_sc=False,
                                             needs_layout_passes=False),
        scratch_types=[
            pltpu.VMEM((fields, nb), jnp.int32),
            pltpu.VMEM((nb, dim), jnp.float32),
            pltpu.VMEM((nb, dim), jnp.float32),
            pltpu.VMEM((dim, fields, nb), jnp.float32),
            pltpu.SemaphoreType.DMA,
            pltpu.SemaphoreType.DMA,
            pltpu.SemaphoreType.DMA,
        ],
    )
    def gather_kernel(idx_hbm, table_hbm, out_hbm, idx_v, gbuf0, gbuf1,
                      slab, gsem0, gsem1, wsem):
        wid = lax.axis_index("s") * _NC + lax.axis_index("c")
        b0 = wid * nb
        pltpu.sync_copy(idx_hbm.at[:, pl.ds(b0, nb)], idx_v)
        gbufs = (gbuf0, gbuf1)
        gsems = (gsem0, gsem1)
        lanes = lax.iota(jnp.int32, 16)

        def start_gather(f):
            pltpu.async_copy(table_hbm.at[idx_v.at[f]], gbufs[f % 2],
                             gsems[f % 2])

        def wait_gather(f):
            pltpu.make_async_copy(table_hbm.at[pl.ds(0, nb)], gbufs[f % 2],
                                  gsems[f % 2]).wait()

        def transpose_field(f):
            gbuf = gbufs[f % 2]

            def g_body(g, _):
                row16 = g * 16 + lanes
                for c in range(dim):
                    col16 = jnp.full((16,), c, dtype=jnp.int32)
                    vals = plsc.load_gather(gbuf, [row16, col16])
                    slab[c, f, pl.ds(g * 16, 16)] = vals
                return 0

            lax.fori_loop(0, nb // 16, g_body, 0)

        start_gather(0)
        for f in range(fields):
            if f + 1 < fields:
                start_gather(f + 1)
            wait_gather(f)
            transpose_field(f)
        for c in range(dim):
            pltpu.async_copy(slab.at[c], out_hbm.at[:, c, pl.ds(b0, nb)],
                             wsem)
        for c in range(dim):
            pltpu.make_async_copy(out_hbm.at[:, c, pl.ds(0, nb)],
                                  slab.at[c], wsem).wait()

    return gather_kernel


def kernel(inputs, embedding):
    batch, fields = inputs.shape
    vocab, dim = embedding.shape
    assert batch % _NW == 0
    idx_t = inputs.T.astype(jnp.int32)
    out_t = _make_gather(fields, batch, dim)(idx_t, embedding)
    return out_t.transpose(2, 0, 1)


# speedup vs baseline: 1.1881x; 1.1108x over previous
import functools

import jax
import jax.numpy as jnp
from jax import lax
from jax.experimental import pallas as pl
from jax.experimental.pallas import tpu as pltpu
from jax.experimental.pallas import tpu_sc as plsc

_NC = 2
_NS = 16
_NW = _NC * _NS


def _make_gather(fields: int, batch: int, dim: int):
    mesh = plsc.VectorSubcoreMesh(core_axis_name="c", subcore_axis_name="s")
    nb = batch // _NW

    @functools.partial(
        pl.kernel,
        mesh=mesh,
        out_type=jax.ShapeDtypeStruct((fields, dim, batch), jnp.float32),
        compiler_params=pltpu.CompilerParams(use_tc_tiling_on_sc=False,
                                             needs_layout_passes=False),
        scratch_types=[
            pltpu.VMEM((fields, nb), jnp.int32),
            pltpu.VMEM((fields * nb, dim), jnp.float32),
            pltpu.VMEM((dim, nb), jnp.float32),
            pltpu.VMEM((dim, nb), jnp.float32),
            pltpu.SemaphoreType.DMA,
            pltpu.SemaphoreType.DMA,
            pltpu.SemaphoreType.DMA,
        ],
    )
    def gather_kernel(idx_hbm, table_hbm, out_hbm, idx_v, rows_v,
                      slab0, slab1, gsem, wsem0, wsem1):
        wid = lax.axis_index("s") * _NC + lax.axis_index("c")
        b0 = wid * nb
        pltpu.sync_copy(idx_hbm.at[:, pl.ds(b0, nb)], idx_v)
        for f in range(fields):
            pltpu.async_copy(table_hbm.at[idx_v.at[f]],
                             rows_v.at[pl.ds(f * nb, nb)], gsem)
        pltpu.make_async_copy(table_hbm.at[pl.ds(0, fields * nb)], rows_v,
                              gsem).wait()
        slabs = (slab0, slab1)
        wsems = (wsem0, wsem1)
        lanes = lax.iota(jnp.int32, 16)
        rots = [jnp.bitwise_and(lanes + k, 15) for k in range(16)]

        def transpose_field(f, slab):
            def g_body(g, _):
                row_d = f * nb + g * 16 + lanes
                for h in range(dim // 16):
                    for k in range(16):
                        col = rots[k] + (h * 16)
                        vals = plsc.load_gather(rows_v, [row_d, col])
                        plsc.store_scatter(slab, [col, g * 16 + lanes], vals)
                return 0

            lax.fori_loop(0, nb // 16, g_body, 0)

        def drain_wb(slab, wsem):
            pltpu.make_async_copy(out_hbm.at[0, :, pl.ds(0, nb)], slab,
                                  wsem).wait()

        def do_field(f, p):
            transpose_field(f, slabs[p])
            pltpu.async_copy(slabs[p], out_hbm.at[f, :, pl.ds(b0, nb)],
                             wsems[p])

        do_field(0, 0)
        do_field(1, 1)

        def pair_body(t, _):
            for q in range(2):
                drain_wb(slabs[q], wsems[q])
                do_field(2 * t + q, q)
            return 0

        lax.fori_loop(1, fields // 2, pair_body, 0)
        drain_wb(slabs[0], wsems[0])
        drain_wb(slabs[1], wsems[1])

    return gather_kernel


def kernel(inputs, embedding):
    batch, fields = inputs.shape
    vocab, dim = embedding.shape
    assert batch % _NW == 0
    idx_t = inputs.T.astype(jnp.int32)
    out_t = _make_gather(fields, batch, dim)(idx_t, embedding)
    return out_t.transpose(2, 0, 1)
